# baseline (speedup 1.0000x reference)
import functools

import jax
import jax.numpy as jnp
from jax import lax
from jax.experimental import pallas as pl
from jax.experimental.pallas import tpu as pltpu

_LAYERS = 3
_CH = 1024
_NV = 9
_T = 2048
_BLK = 1024


def _lin(a, w, b):
    out = lax.dot_general(a, w, (((1,), (1,)), ((), ())),
                          preferred_element_type=jnp.float32)
    return out + b


_HALVES = 2
_HB = _BLK // _HALVES


def _body(x_ref, netw0_ref, netw_hbm, netb_ref, rw_hbm, rb_ref, rwo_ref,
          rbo_ref, h_ref, routes_ref, idx_ref, nwv, rwv, sem_n1, sem_n, sem_r):
    pid = pl.program_id(0)

    @pl.when(pid == 0)
    def _start_w():
        pltpu.make_async_copy(netw_hbm.at[1:2], nwv.at[0:1], sem_n1).start()
        pltpu.make_async_copy(netw_hbm.at[2:], nwv.at[1:], sem_n).start()
        pltpu.make_async_copy(rw_hbm, rwv, sem_r).start()

    hs = [x_ref[pl.ds(k * _HB, _HB), :] for k in range(_HALVES)]
    hs = [_lin(h, netw0_ref[0], netb_ref[0]) for h in hs]
    hs = [jnp.maximum(h, 0.2 * h) for h in hs]

    @pl.when(pid == 0)
    def _wait_nw1():
        pltpu.make_async_copy(netw_hbm.at[1:2], nwv.at[0:1], sem_n1).wait()

    hs = [_lin(h, nwv[0], netb_ref[1]) for h in hs]
    hs = [jnp.maximum(h, 0.2 * h) for h in hs]

    @pl.when(pid == 0)
    def _wait_nw():
        pltpu.make_async_copy(netw_hbm.at[2:], nwv.at[1:], sem_n).wait()

    hs = [_lin(h, nwv[1], netb_ref[2]) for h in hs]
    hs = [jnp.maximum(h, 0.2 * h) for h in hs]
    hs = [_lin(h, nwv[2], netb_ref[_LAYERS]) for h in hs]
    for k in range(_HALVES):
        h_ref[pl.ds(k * _HB, _HB), :] = hs[k]

    @pl.when(pid == 0)
    def _wait_rw():
        pltpu.make_async_copy(rw_hbm, rwv, sem_r).wait()

    gs = hs
    for l in range(_LAYERS):
        gs = [_lin(g, rwv[l], rb_ref[l]) for g in gs]
        gs = [jnp.maximum(g, 0.2 * g) for g in gs]
    for k in range(_HALVES):
        logits = _lin(gs[k], rwo_ref[...], rbo_ref[...])

        m = jnp.max(logits, axis=1, keepdims=True)
        e = jnp.exp(logits - m)
        routes = e / jnp.sum(e, axis=1, keepdims=True)
        routes_ref[pl.ds(k * _HB, _HB), :] = jnp.pad(routes,
                                                     ((0, 0), (0, 128 - _NV)))

        mx = jnp.max(routes, axis=1, keepdims=True)
        iot = lax.broadcasted_iota(jnp.int32, (_HB, _NV), 1)
        idx = jnp.min(jnp.where(routes == mx, iot, _NV), axis=1)
        idx_ref[pl.ds(k * _HB, _HB)] = idx


@functools.partial(jax.jit)
def _run(xs, net_W, net_b, r_W, r_b, r_W_out, r_b_out2):
    grid = (_T // _BLK,)
    return pl.pallas_call(
        _body,
        grid=grid,
        in_specs=[
            pl.BlockSpec((_BLK, _CH), lambda i: (i, 0)),
            pl.BlockSpec((1, _CH, _CH), lambda i: (0, 0, 0)),
            pl.BlockSpec(memory_space=pl.ANY),
            pl.BlockSpec((_LAYERS + 1, _CH), lambda i: (0, 0)),
            pl.BlockSpec(memory_space=pl.ANY),
            pl.BlockSpec((_LAYERS, _CH), lambda i: (0, 0)),
            pl.BlockSpec((_NV, _CH), lambda i: (0, 0)),
            pl.BlockSpec((1, _NV), lambda i: (0, 0)),
        ],
        out_specs=[
            pl.BlockSpec((_BLK, _CH), lambda i: (i, 0)),
            pl.BlockSpec((_BLK, 128), lambda i: (i, 0)),
            pl.BlockSpec((_BLK,), lambda i: (i,)),
        ],
        out_shape=[
            jax.ShapeDtypeStruct((_T, _CH), jnp.float32),
            jax.ShapeDtypeStruct((_T, 128), jnp.float32),
            jax.ShapeDtypeStruct((_T,), jnp.int32),
        ],
        scratch_shapes=[
            pltpu.VMEM((_LAYERS, _CH, _CH), jnp.float32),
            pltpu.VMEM((_LAYERS, _CH, _CH), jnp.float32),
            pltpu.SemaphoreType.DMA,
            pltpu.SemaphoreType.DMA,
            pltpu.SemaphoreType.DMA,
        ],
    )(xs, net_W, net_W, net_b, r_W, r_b, r_W_out, r_b_out2)


def kernel(x, net_W, net_b, r_W, r_b, r_W_out, r_b_out):
    batch, time, channels = x.shape
    xs = x.reshape(time, channels)
    h, routes_pad, idx = _run(xs, net_W, net_b, r_W, r_b, r_W_out,
                              r_b_out.reshape(1, -1))
    return h, routes_pad[:, :_NV], idx


# speedup vs baseline: 1.0768x; 1.0768x over previous
import functools

import jax
import jax.numpy as jnp
from jax import lax
from jax.experimental import pallas as pl
from jax.experimental.pallas import tpu as pltpu

_LAYERS = 3
_CH = 1024
_NV = 9
_T = 2048
_BLK = 1024


def _lin(a, w, b):
    out = lax.dot_general(a, w, (((1,), (1,)), ((), ())),
                          preferred_element_type=jnp.float32)
    return out + b


_HALVES = 2
_HB = _BLK // _HALVES


def _body(x_ref, netw0_ref, netw_hbm, netb_ref, rw_hbm, rb_ref, rwo_ref,
          rbo_ref, h_ref, routes_ref, idx_ref, nwv, rwv, sem_n1, sem_n, sem_r):
    pid = pl.program_id(0)

    @pl.when(pid == 0)
    def _start_w():
        pltpu.make_async_copy(netw_hbm.at[1:2], nwv.at[0:1], sem_n1).start()
        pltpu.make_async_copy(netw_hbm.at[2:], nwv.at[1:], sem_n).start()
        pltpu.make_async_copy(rw_hbm, rwv, sem_r).start()

    hs = [x_ref[pl.ds(k * _HB, _HB), :] for k in range(_HALVES)]
    hs = [_lin(h, netw0_ref[0], netb_ref[0]) for h in hs]
    hs = [jnp.where(h >= 0, h, 0.2 * h) for h in hs]

    @pl.when(pid == 0)
    def _wait_nw1():
        pltpu.make_async_copy(netw_hbm.at[1:2], nwv.at[0:1], sem_n1).wait()

    hs = [_lin(h, nwv[0], netb_ref[1]) for h in hs]
    hs = [jnp.where(h >= 0, h, 0.2 * h) for h in hs]

    @pl.when(pid == 0)
    def _wait_nw():
        pltpu.make_async_copy(netw_hbm.at[2:], nwv.at[1:], sem_n).wait()

    hs = [_lin(h, nwv[1], netb_ref[2]) for h in hs]
    hs = [jnp.where(h >= 0, h, 0.2 * h) for h in hs]
    hs = [_lin(h, nwv[2], netb_ref[_LAYERS]) for h in hs]
    for k in range(_HALVES):
        h_ref[pl.ds(k * _HB, _HB), :] = hs[k]

    @pl.when(pid == 0)
    def _wait_rw():
        pltpu.make_async_copy(rw_hbm, rwv, sem_r).wait()

    gs = hs
    for l in range(_LAYERS):
        gs = [_lin(g, rwv[l], rb_ref[l]) for g in gs]
        gs = [jnp.where(g >= 0, g, 0.2 * g) for g in gs]
    for k in range(_HALVES):
        logits = _lin(gs[k], rwo_ref[...], rbo_ref[...])

        m = jnp.max(logits, axis=1, keepdims=True)
        e = jnp.exp(logits - m)
        routes = e / jnp.sum(e, axis=1, keepdims=True)
        routes_ref[pl.ds(k * _HB, _HB), :] = jnp.pad(routes,
                                                     ((0, 0), (0, 128 - _NV)))

        mx = jnp.max(routes, axis=1, keepdims=True)
        iot = lax.broadcasted_iota(jnp.int32, (_HB, _NV), 1)
        idx = jnp.min(jnp.where(routes == mx, iot, _NV), axis=1)
        idx_ref[pl.ds(k * _HB, _HB)] = idx


@functools.partial(jax.jit)
def _run(xs, net_W, net_b, r_W, r_b, r_W_out, r_b_out2):
    grid = (_T // _BLK,)
    return pl.pallas_call(
        _body,
        grid=grid,
        in_specs=[
            pl.BlockSpec((_BLK, _CH), lambda i: (i, 0)),
            pl.BlockSpec((1, _CH, _CH), lambda i: (0, 0, 0)),
            pl.BlockSpec(memory_space=pl.ANY),
            pl.BlockSpec((_LAYERS + 1, _CH), lambda i: (0, 0)),
            pl.BlockSpec(memory_space=pl.ANY),
            pl.BlockSpec((_LAYERS, _CH), lambda i: (0, 0)),
            pl.BlockSpec((_NV, _CH), lambda i: (0, 0)),
            pl.BlockSpec((1, _NV), lambda i: (0, 0)),
        ],
        out_specs=[
            pl.BlockSpec((_BLK, _CH), lambda i: (i, 0)),
            pl.BlockSpec((_BLK, 128), lambda i: (i, 0)),
            pl.BlockSpec((_BLK,), lambda i: (i,)),
        ],
        out_shape=[
            jax.ShapeDtypeStruct((_T, _CH), jnp.float32),
            jax.ShapeDtypeStruct((_T, 128), jnp.float32),
            jax.ShapeDtypeStruct((_T,), jnp.int32),
        ],
        scratch_shapes=[
            pltpu.VMEM((_LAYERS, _CH, _CH), jnp.float32),
            pltpu.VMEM((_LAYERS, _CH, _CH), jnp.float32),
            pltpu.SemaphoreType.DMA,
            pltpu.SemaphoreType.DMA,
            pltpu.SemaphoreType.DMA,
        ],
    )(xs, net_W, net_W, net_b, r_W, r_b, r_W_out, r_b_out2)


def kernel(x, net_W, net_b, r_W, r_b, r_W_out, r_b_out):
    batch, time, channels = x.shape
    xs = x.reshape(time, channels)
    h, routes_pad, idx = _run(xs, net_W, net_b, r_W, r_b, r_W_out,
                              r_b_out.reshape(1, -1))
    return h, routes_pad[:, :_NV], idx
